# baseline (speedup 1.0000x reference)
import functools

import jax
import jax.numpy as jnp
from jax import lax
from jax.experimental import pallas as pl
from jax.experimental.pallas import tpu as pltpu
from jax.experimental.pallas import tpu_sc as plsc

_NUM_CORES = 2
_NUM_SUBCORES = 16
_NUM_WORKERS = _NUM_CORES * _NUM_SUBCORES
_CHUNK = 128
_K = 2
_TCOLS = 20480


def _transpose_kernel(wt_a, wt_b, out_ref):
    out_ref[:, 0:64] = jnp.swapaxes(wt_a[...], 0, 1)
    out_ref[:, 64:128] = jnp.swapaxes(wt_b[...], 0, 1)


def _stage_table(weight):
    v, d = weight.shape
    wt = weight.T
    nb = (v + 2 * _TCOLS - 1) // (2 * _TCOLS)
    maxb = (v + _TCOLS - 1) // _TCOLS - 1
    h = nb * _TCOLS
    staged = pl.pallas_call(
        _transpose_kernel,
        grid=(nb,),
        in_specs=[
            pl.BlockSpec((d, _TCOLS), lambda j: (0, j)),
            pl.BlockSpec((d, _TCOLS), lambda j: (0, jnp.minimum(j + nb, maxb))),
        ],
        out_specs=pl.BlockSpec((_TCOLS, 2 * d), lambda j: (j, 0)),
        out_shape=jax.ShapeDtypeStruct((h, 2 * d), jnp.float32),
    )(wt, wt)
    return staged, h


def _emb_kernel(n_chunks, tok_hbm, w_hbm, out_hbm, idx_v, rows_v, g0, g1, s0, s1):
    wid = lax.axis_index("s") * _NUM_CORES + lax.axis_index("c")
    n_groups = n_chunks // _K
    base = wid * n_chunks
    gsem = (g0, g1)
    ssem = (s0, s1)
    pltpu.sync_copy(tok_hbm.at[wid], idx_v)

    def fire_group(j, b):
        return [
            pltpu.async_copy(
                w_hbm.at[idx_v.at[j * _K + i]], rows_v.at[b, i], gsem[b]
            )
            for i in range(_K)
        ]

    gathers = {}
    stores = {}
    gathers[0] = fire_group(0, 0)
    for j in range(n_groups):
        b = j % 2
        for c in gathers[j]:
            c.wait()
        stores[j] = pltpu.async_copy(
            rows_v.at[b], out_hbm.at[pl.ds(base + j * _K, _K), :, 0:64], ssem[b]
        )
        if j >= 1:
            stores[j - 1].wait()
        if j + 1 < n_groups:
            gathers[j + 1] = fire_group(j + 1, 1 - b)
    stores[n_groups - 1].wait()


def kernel(token_ids, weight):
    b, l = token_ids.shape
    v, d = weight.shape
    n = b * l
    per_w = n // _NUM_WORKERS
    n_chunks = per_w // _CHUNK
    assert per_w * _NUM_WORKERS == n and n_chunks * _CHUNK == per_w
    assert n_chunks % _K == 0

    staged, h = _stage_table(weight)
    rows = staged.reshape(2 * h, d)
    tokf = token_ids.reshape(-1).astype(jnp.int32)
    idx2 = jnp.where(tokf < h, 2 * tokf, 2 * tokf - (2 * h - 1))
    tok = idx2.reshape(_NUM_WORKERS, n_chunks, _CHUNK)
    mesh = plsc.VectorSubcoreMesh(
        core_axis_name="c",
        subcore_axis_name="s",
        num_cores=_NUM_CORES,
        num_subcores=_NUM_SUBCORES,
    )
    run = functools.partial(
        pl.kernel,
        mesh=mesh,
        compiler_params=pltpu.CompilerParams(use_tc_tiling_on_sc=False),
        out_type=jax.ShapeDtypeStruct((n // _CHUNK, _CHUNK, 128), jnp.float32),
        scratch_types=[
            pltpu.VMEM((n_chunks, _CHUNK), jnp.int32),
            pltpu.VMEM((2, _K, _CHUNK, d), jnp.float32),
            pltpu.SemaphoreType.DMA,
            pltpu.SemaphoreType.DMA,
            pltpu.SemaphoreType.DMA,
            pltpu.SemaphoreType.DMA,
        ],
    )(functools.partial(_emb_kernel, n_chunks))
    out = run(tok, rows)
    return out.reshape(n, 128)[:, 0:d].reshape(b, l, d)


# speedup vs baseline: 1.0355x; 1.0355x over previous
import functools

import jax
import jax.numpy as jnp
from jax import lax
from jax.experimental import pallas as pl
from jax.experimental.pallas import tpu as pltpu
from jax.experimental.pallas import tpu_sc as plsc

_NUM_CORES = 2
_NUM_SUBCORES = 16
_NUM_WORKERS = _NUM_CORES * _NUM_SUBCORES
_CHUNK = 128
_K = 5
_TCOLS = 20480


def _transpose_kernel(wt_ref, out_ref):
    out_ref[:, 0:64] = jnp.swapaxes(wt_ref[:, 0:_TCOLS], 0, 1)
    out_ref[:, 64:128] = jnp.swapaxes(wt_ref[:, _TCOLS:], 0, 1)


def _stage_table(weight):
    v, d = weight.shape
    wt = weight.T
    nb = (v + 2 * _TCOLS - 1) // (2 * _TCOLS)
    h = nb * _TCOLS
    staged = pl.pallas_call(
        _transpose_kernel,
        grid=(nb,),
        in_specs=[pl.BlockSpec((d, 2 * _TCOLS), lambda j: (0, j))],
        out_specs=pl.BlockSpec((_TCOLS, 2 * d), lambda j: (j, 0)),
        out_shape=jax.ShapeDtypeStruct((h, 2 * d), jnp.float32),
    )(wt)
    return staged, h


def _emb_kernel(n_chunks, tok_hbm, w_hbm, out_hbm, idx_v, rows_v, g0, g1, s0, s1):
    wid = lax.axis_index("s") * _NUM_CORES + lax.axis_index("c")
    n_groups = n_chunks // _K
    base = wid * n_chunks
    gsem = (g0, g1)
    ssem = (s0, s1)
    pltpu.sync_copy(tok_hbm.at[wid], idx_v)

    def fire_group(j, b):
        return [
            pltpu.async_copy(
                w_hbm.at[idx_v.at[j * _K + i]], rows_v.at[b, i], gsem[b]
            )
            for i in range(_K)
        ]

    gathers = {}
    stores = {}
    gathers[0] = fire_group(0, 0)
    for j in range(n_groups):
        b = j % 2
        for c in gathers[j]:
            c.wait()
        stores[j] = pltpu.async_copy(
            rows_v.at[b], out_hbm.at[pl.ds(base + j * _K, _K), :, 0:64], ssem[b]
        )
        if j >= 1:
            stores[j - 1].wait()
        if j + 1 < n_groups:
            gathers[j + 1] = fire_group(j + 1, 1 - b)
    stores[n_groups - 1].wait()


def kernel(token_ids, weight):
    b, l = token_ids.shape
    v, d = weight.shape
    n = b * l
    per_w = n // _NUM_WORKERS
    n_chunks = per_w // _CHUNK
    assert per_w * _NUM_WORKERS == n and n_chunks * _CHUNK == per_w
    assert n_chunks % _K == 0

    staged, h = _stage_table(weight)
    rows = staged.reshape(2 * h, d)
    tokf = token_ids.reshape(-1).astype(jnp.int32)
    r = tokf & (_TCOLS - 1)
    half = (tokf // _TCOLS) & 1
    pair = tokf // (2 * _TCOLS)
    idx2 = pair * (2 * _TCOLS) + 2 * r + half
    tok = idx2.reshape(_NUM_WORKERS, n_chunks, _CHUNK)
    mesh = plsc.VectorSubcoreMesh(
        core_axis_name="c",
        subcore_axis_name="s",
        num_cores=_NUM_CORES,
        num_subcores=_NUM_SUBCORES,
    )
    run = functools.partial(
        pl.kernel,
        mesh=mesh,
        compiler_params=pltpu.CompilerParams(use_tc_tiling_on_sc=False),
        out_type=jax.ShapeDtypeStruct((n // _CHUNK, _CHUNK, 128), jnp.float32),
        scratch_types=[
            pltpu.VMEM((n_chunks, _CHUNK), jnp.int32),
            pltpu.VMEM((2, _K, _CHUNK, d), jnp.float32),
            pltpu.SemaphoreType.DMA,
            pltpu.SemaphoreType.DMA,
            pltpu.SemaphoreType.DMA,
            pltpu.SemaphoreType.DMA,
        ],
    )(functools.partial(_emb_kernel, n_chunks))
    out = run(tok, rows)
    return out.reshape(n, 128)[:, 0:d].reshape(b, l, d)


# speedup vs baseline: 1.0358x; 1.0002x over previous
import functools

import jax
import jax.numpy as jnp
from jax import lax
from jax.experimental import pallas as pl
from jax.experimental.pallas import tpu as pltpu
from jax.experimental.pallas import tpu_sc as plsc

_NUM_CORES = 2
_NUM_SUBCORES = 16
_NUM_WORKERS = _NUM_CORES * _NUM_SUBCORES
_CHUNK = 128
_K = 5
_TCOLS = 16384


def _transpose_kernel(wt_ref, out_ref):
    out_ref[:, 0:64] = jnp.swapaxes(wt_ref[:, 0:_TCOLS], 0, 1)
    out_ref[:, 64:128] = jnp.swapaxes(wt_ref[:, _TCOLS:], 0, 1)


def _stage_table(weight):
    v, d = weight.shape
    wt = weight.T
    nb = (v + 2 * _TCOLS - 1) // (2 * _TCOLS)
    h = nb * _TCOLS
    staged = pl.pallas_call(
        _transpose_kernel,
        grid=(nb,),
        in_specs=[pl.BlockSpec((d, 2 * _TCOLS), lambda j: (0, j))],
        out_specs=pl.BlockSpec((_TCOLS, 2 * d), lambda j: (j, 0)),
        out_shape=jax.ShapeDtypeStruct((h, 2 * d), jnp.float32),
    )(wt)
    return staged, h


def _emb_kernel(n_chunks, tok_hbm, w_hbm, out_hbm, idx_v, rows_v, g0, g1, s0, s1):
    wid = lax.axis_index("s") * _NUM_CORES + lax.axis_index("c")
    n_groups = n_chunks // _K
    base = wid * n_chunks
    gsem = (g0, g1)
    ssem = (s0, s1)
    pltpu.sync_copy(tok_hbm.at[wid], idx_v)

    def fire_group(j, b):
        return [
            pltpu.async_copy(
                w_hbm.at[idx_v.at[j * _K + i]], rows_v.at[b, i], gsem[b]
            )
            for i in range(_K)
        ]

    gathers = {}
    stores = {}
    gathers[0] = fire_group(0, 0)
    for j in range(n_groups):
        b = j % 2
        for c in gathers[j]:
            c.wait()
        stores[j] = pltpu.async_copy(
            rows_v.at[b], out_hbm.at[pl.ds(base + j * _K, _K), :, 0:64], ssem[b]
        )
        if j >= 1:
            stores[j - 1].wait()
        if j + 1 < n_groups:
            gathers[j + 1] = fire_group(j + 1, 1 - b)
    stores[n_groups - 1].wait()


def kernel(token_ids, weight):
    b, l = token_ids.shape
    v, d = weight.shape
    n = b * l
    per_w = n // _NUM_WORKERS
    n_chunks = per_w // _CHUNK
    assert per_w * _NUM_WORKERS == n and n_chunks * _CHUNK == per_w
    assert n_chunks % _K == 0

    staged, h = _stage_table(weight)
    rows = staged.reshape(2 * h, d)
    tokf = token_ids.reshape(-1).astype(jnp.int32)
    r = tokf & (_TCOLS - 1)
    half = (tokf // _TCOLS) & 1
    pair = tokf // (2 * _TCOLS)
    idx2 = pair * (2 * _TCOLS) + 2 * r + half
    tok = idx2.reshape(_NUM_WORKERS, n_chunks, _CHUNK)
    mesh = plsc.VectorSubcoreMesh(
        core_axis_name="c",
        subcore_axis_name="s",
        num_cores=_NUM_CORES,
        num_subcores=_NUM_SUBCORES,
    )
    run = functools.partial(
        pl.kernel,
        mesh=mesh,
        compiler_params=pltpu.CompilerParams(use_tc_tiling_on_sc=False),
        out_type=jax.ShapeDtypeStruct((n // _CHUNK, _CHUNK, 128), jnp.float32),
        scratch_types=[
            pltpu.VMEM((n_chunks, _CHUNK), jnp.int32),
            pltpu.VMEM((2, _K, _CHUNK, d), jnp.float32),
            pltpu.SemaphoreType.DMA,
            pltpu.SemaphoreType.DMA,
            pltpu.SemaphoreType.DMA,
            pltpu.SemaphoreType.DMA,
        ],
    )(functools.partial(_emb_kernel, n_chunks))
    out = run(tok, rows)
    return out.reshape(n, 128)[:, 0:d].reshape(b, l, d)
